# baseline (speedup 1.0000x reference)
import functools

import jax
import jax.numpy as jnp
from jax import lax
from jax.experimental import pallas as pl
from jax.experimental.pallas import tpu as pltpu
from jax.experimental.pallas import tpu_sc as plsc

EMB_D = 64
PAD_D = 128
SCALE = float(EMB_D) ** 0.5
LANES = 16
NBUF = 2
CHUNK = 256


@functools.partial(jax.jit, static_argnames=("b_total", "h_total"))
def _lookup(x_flat, table_p, b_total, h_total):
    n_total = x_flat.shape[0]
    info = plsc.get_sparse_core_info()
    nw = info.num_cores * info.num_subcores
    b_per_w = b_total // nw
    n_chunks = h_total * (b_per_w // CHUNK)
    sub_per_h = b_per_w // CHUNK
    assert b_per_w % CHUNK == 0 and b_total % nw == 0

    mesh = plsc.VectorSubcoreMesh(core_axis_name="c", subcore_axis_name="s")

    @functools.partial(
        pl.kernel,
        mesh=mesh,
        out_type=jax.ShapeDtypeStruct((h_total, EMB_D, b_total), jnp.float32),
        scratch_types=[
            [pltpu.VMEM((CHUNK,), jnp.int32) for _ in range(NBUF)],
            [pltpu.VMEM((CHUNK, PAD_D), jnp.float32) for _ in range(NBUF)],
            [pltpu.VMEM((8, 128), jnp.float32) for _ in range(NBUF)],
            [pltpu.SemaphoreType.DMA for _ in range(NBUF)],
            [pltpu.SemaphoreType.DMA for _ in range(NBUF)],
        ],
        compiler_params=pltpu.CompilerParams(
            use_tc_tiling_on_sc=True, needs_layout_passes=False
        ),
    )
    def k(x_hbm, table_hbm, out_hbm, idx_v, rows, tbuf, sem_g, sem_t):
        wid = lax.axis_index("s") * info.num_cores + lax.axis_index("c")
        wb = wid * b_per_w
        biota = lax.iota(jnp.int32, LANES)

        def x_off(c):
            h = c // sub_per_h
            return h * b_total + wb + (c % sub_per_h) * CHUNK

        pltpu.sync_copy(x_hbm.at[pl.ds(x_off(0), CHUNK)], idx_v[0])
        pltpu.async_copy(table_hbm.at[idx_v[0]], rows[0], sem_g[0])

        @pl.loop(0, n_chunks, step=NBUF)
        def _chunk_loop(c0):
            for b in range(NBUF):
                c = c0 + b
                nb = (b + 1) % NBUF
                nxt = c + 1
                h = c // sub_per_h
                bb = wb + (c % sub_per_h) * CHUNK

                @pl.when(nxt < n_chunks)
                def _issue_next_gather():
                    pltpu.sync_copy(
                        x_hbm.at[pl.ds(x_off(nxt), CHUNK)], idx_v[nb]
                    )
                    pltpu.async_copy(
                        table_hbm.at[idx_v[nb]], rows[nb], sem_g[nb]
                    )

                pltpu.make_async_copy(
                    table_hbm.at[idx_v[b]], rows[b], sem_g[b]
                ).wait()

                for t in range(16):
                    i, j = divmod(t, 2)
                    tb = t % NBUF

                    def _wait_tile_dma():
                        pltpu.make_async_copy(
                            tbuf[tb],
                            out_hbm.at[
                                h, pl.ds(8 * i, 8), pl.ds(bb + 128 * j, 128)
                            ],
                            sem_t[tb],
                        ).wait()

                    if t >= NBUF:
                        _wait_tile_dma()
                    else:
                        pl.when(c > 0)(_wait_tile_dma)

                    for s in range(8):
                        d = 8 * i + s
                        dcol = jnp.full((LANES,), d, jnp.int32)

                        @pl.loop(0, 8)
                        def _gth(g):
                            bidx = biota + (128 * j + g * 16)
                            vals = plsc.load_gather(rows[b], [bidx, dcol])
                            tbuf[tb][s, pl.ds(g * 16, 16)] = vals * SCALE

                    pltpu.async_copy(
                        tbuf[tb],
                        out_hbm.at[
                            h, pl.ds(8 * i, 8), pl.ds(bb + 128 * j, 128)
                        ],
                        sem_t[tb],
                    )

        lastc = n_chunks - 1
        lh = lastc // sub_per_h
        lbb = wb + (lastc % sub_per_h) * CHUNK
        for t in range(16 - NBUF, 16):
            i, j = divmod(t, 2)
            pltpu.make_async_copy(
                tbuf[t % NBUF],
                out_hbm.at[lh, pl.ds(8 * i, 8), pl.ds(lbb + 128 * j, 128)],
                sem_t[t % NBUF],
            ).wait()

    return k(x_flat, table_p)


def kernel(x, table):
    b, h = x.shape
    x_flat = x.T.reshape(-1)
    table_p = jnp.pad(table, ((0, 0), (0, PAD_D - EMB_D)))
    out = _lookup(x_flat, table_p, b, h)
    return out.transpose(2, 0, 1)
